# baseline (speedup 1.0000x reference)
import functools

import jax
import jax.numpy as jnp
from jax import lax
from jax.experimental import pallas as pl
from jax.experimental.pallas import tpu as pltpu
from jax.experimental.pallas import tpu_sc as plsc

_B = 16384
_DM = 32
_DS = 16
_W = 128
_NC = 2
_NS = 16
_NW = _NC * _NS
_BPW = _B // _NW
_CHUNK = 128
_NCH = _BPW // _CHUNK

_mesh = plsc.VectorSubcoreMesh(core_axis_name="c", subcore_axis_name="s")


@functools.partial(
    pl.kernel,
    out_type=jax.ShapeDtypeStruct((_B, _W), jnp.float32),
    mesh=_mesh,
    scratch_types=[
        pltpu.VMEM((_NCH, _CHUNK), jnp.int32),
        pltpu.VMEM((_NCH, _CHUNK), jnp.int32),
        pltpu.VMEM((_CHUNK, _DM), jnp.float32),
        pltpu.VMEM((_CHUNK, _DS), jnp.float32),
        pltpu.VMEM((_CHUNK, _W), jnp.float32),
        pltpu.SemaphoreType.DMA,
        pltpu.SemaphoreType.DMA,
        pltpu.SemaphoreType.DMA,
    ],
    compiler_params=pltpu.CompilerParams(use_tc_tiling_on_sc=False),
)
def _gather_kernel(mid_hbm, sid_hbm, wm_hbm, ws_hbm, out_hbm,
                   idx_m, idx_s, mat_v, spc_v, out_v, sem_m, sem_s, sem_o):
    wid = lax.axis_index("s") * _NC + lax.axis_index("c")
    base = wid * _BPW
    pltpu.sync_copy(mid_hbm.at[wid], idx_m)
    pltpu.sync_copy(sid_hbm.at[wid], idx_s)
    for j in range(_NCH):
        cm = pltpu.async_copy(wm_hbm.at[idx_m.at[j]], mat_v, sem_m)
        cs = pltpu.async_copy(ws_hbm.at[idx_s.at[j]], spc_v, sem_s)
        cm.wait()
        cs.wait()
        for i in range(_CHUNK):
            out_v[i, pl.ds(0, 16)] = mat_v[i, pl.ds(0, 16)]
            out_v[i, pl.ds(16, 16)] = mat_v[i, pl.ds(16, 16)]
            out_v[i, pl.ds(_DM, _DS)] = spc_v[i, pl.ds(0, _DS)]
        pltpu.sync_copy(out_v, out_hbm.at[pl.ds(base + j * _CHUNK, _CHUNK)])


def kernel(material_id, space_group, W_material, W_space):
    mid = material_id.astype(jnp.int32).reshape(_NW, _NCH, _CHUNK)
    sid = space_group.astype(jnp.int32).reshape(_NW, _NCH, _CHUNK)
    out = _gather_kernel(mid, sid, W_material, W_space)
    return out[:, : _DM + _DS]


# speedup vs baseline: 2.1375x; 2.1375x over previous
import functools

import jax
import jax.numpy as jnp
from jax import lax
from jax.experimental import pallas as pl
from jax.experimental.pallas import tpu as pltpu
from jax.experimental.pallas import tpu_sc as plsc

_B = 16384
_DM = 32
_DS = 16
_D = _DM + _DS
_NC = 2
_NS = 16
_NW = _NC * _NS
_BPW = _B // _NW
_G = 8
_NG = _BPW // _G

_mesh = plsc.VectorSubcoreMesh(core_axis_name="c", subcore_axis_name="s")


@functools.partial(
    pl.kernel,
    out_type=jax.ShapeDtypeStruct((_B, _D), jnp.float32),
    mesh=_mesh,
    scratch_types=[
        pltpu.VMEM((_BPW + 8,), jnp.int32),
        pltpu.VMEM((_BPW + 8,), jnp.int32),
        pltpu.VMEM((2, _G, 8, _DM), jnp.float32),
        pltpu.VMEM((2, _G, 8, _DS), jnp.float32),
        pltpu.VMEM((2, _G, _D), jnp.float32),
        pltpu.SemaphoreType.DMA,
        pltpu.SemaphoreType.DMA,
        pltpu.SemaphoreType.DMA,
    ],
)
def _gather_kernel(mid_hbm, sid_hbm, wm_hbm, ws_hbm, out_hbm,
                   idx_m, idx_s, blk_m, blk_s, rows_v,
                   sem_m, sem_s, sem_o):
    wid = lax.axis_index("s") * _NC + lax.axis_index("c")
    base = wid * _BPW
    pltpu.sync_copy(mid_hbm.at[pl.ds(base, _BPW)], idx_m.at[pl.ds(0, _BPW)])
    pltpu.sync_copy(sid_hbm.at[pl.ds(base, _BPW)], idx_s.at[pl.ds(0, _BPW)])

    def fire(g, slot):
        tm_vec = jax.lax.shift_right_logical(idx_m[pl.ds(g * _G, 16)], 3)
        ts_vec = jax.lax.shift_right_logical(idx_s[pl.ds(g * _G, 16)], 3)
        for k in range(_G):
            pltpu.async_copy(wm_hbm.at[tm_vec[k]], blk_m.at[slot, k], sem_m)
            pltpu.async_copy(ws_hbm.at[ts_vec[k]], blk_s.at[slot, k], sem_s)

    def drain_extract(g, slot):
        rm_vec = jax.lax.bitwise_and(idx_m[pl.ds(g * _G, 16)], 7)
        rs_vec = jax.lax.bitwise_and(idx_s[pl.ds(g * _G, 16)], 7)
        pltpu.make_async_copy(wm_hbm.at[pl.ds(0, _G)], blk_m.at[slot], sem_m).wait()
        pltpu.make_async_copy(ws_hbm.at[pl.ds(0, _G)], blk_s.at[slot], sem_s).wait()
        for k in range(_G):
            rm = rm_vec[k]
            rs = rs_vec[k]
            rows_v[slot, k, pl.ds(0, 16)] = blk_m[slot, k, rm, pl.ds(0, 16)]
            rows_v[slot, k, pl.ds(16, 16)] = blk_m[slot, k, rm, pl.ds(16, 16)]
            rows_v[slot, k, pl.ds(_DM, 16)] = blk_s[slot, k, rs, pl.ds(0, 16)]
        pltpu.async_copy(rows_v.at[slot], out_hbm.at[pl.ds(base + g * _G, _G)], sem_o)

    fire(0, 0)

    @pl.loop(0, _NG)
    def _body(c):
        slot = jax.lax.bitwise_and(c, 1)
        nslot = jax.lax.bitwise_and(c + 1, 1)

        @pl.when(c + 1 < _NG)
        def _():
            fire(c + 1, nslot)

        @pl.when(c >= 2)
        def _():
            pltpu.make_async_copy(
                rows_v.at[slot], out_hbm.at[pl.ds(base, _G)], sem_o).wait()

        drain_extract(c, slot)

    pltpu.make_async_copy(rows_v.at[0], out_hbm.at[pl.ds(base, _G)], sem_o).wait()
    pltpu.make_async_copy(rows_v.at[1], out_hbm.at[pl.ds(base, _G)], sem_o).wait()


def kernel(material_id, space_group, W_material, W_space):
    wm3 = W_material.reshape(125000, 8, _DM)
    ws3 = W_space.reshape(12500, 8, _DS)
    return _gather_kernel(material_id.astype(jnp.int32),
                          space_group.astype(jnp.int32), wm3, ws3)


# speedup vs baseline: 3.4647x; 1.6209x over previous
import functools

import jax
import jax.numpy as jnp
from jax import lax
from jax.experimental import pallas as pl
from jax.experimental.pallas import tpu as pltpu
from jax.experimental.pallas import tpu_sc as plsc

_B = 16384
_DM = 32
_DS = 16
_D = _DM + _DS
_NC = 2
_NS = 16
_NW = _NC * _NS
_BPW = _B // _NW
_G = 8
_NG = _BPW // _G

_mesh = plsc.VectorSubcoreMesh(core_axis_name="c", subcore_axis_name="s")


@functools.partial(
    pl.kernel,
    out_type=jax.ShapeDtypeStruct((_B, _D), jnp.float32),
    mesh=_mesh,
    scratch_types=[
        pltpu.VMEM((_BPW + 8,), jnp.int32),
        pltpu.VMEM((_BPW + 8,), jnp.int32),
        pltpu.VMEM((2, _G, _DM, 128), jnp.float32),
        pltpu.VMEM((2, _G, 8, _DS), jnp.float32),
        pltpu.VMEM((2, _G, _D), jnp.float32),
        pltpu.SemaphoreType.DMA,
        pltpu.SemaphoreType.DMA,
        pltpu.SemaphoreType.DMA,
    ],
    compiler_params=pltpu.CompilerParams(needs_layout_passes=False),
)
def _gather_kernel(mid_hbm, sid_hbm, wmt_hbm, ws_hbm, out_hbm,
                   idx_m, idx_s, blk_m, blk_s, rows_v,
                   sem_m, sem_s, sem_o):
    wid = lax.axis_index("s") * _NC + lax.axis_index("c")
    base = wid * _BPW
    pltpu.sync_copy(mid_hbm.at[pl.ds(base, _BPW)], idx_m.at[pl.ds(0, _BPW)])
    pltpu.sync_copy(sid_hbm.at[pl.ds(base, _BPW)], idx_s.at[pl.ds(0, _BPW)])

    def fire(g, slot):
        cm_vec = jax.lax.shift_right_logical(idx_m[pl.ds(g * _G, 16)], 7)
        ts_vec = jax.lax.shift_right_logical(idx_s[pl.ds(g * _G, 16)], 3)
        for k in range(_G):
            off = pl.multiple_of(cm_vec[k] * 128, 128)
            pltpu.async_copy(
                wmt_hbm.at[:, pl.ds(off, 128)], blk_m.at[slot, k], sem_m)
            pltpu.async_copy(ws_hbm.at[ts_vec[k]], blk_s.at[slot, k], sem_s)

    iota16 = lax.iota(jnp.int32, 16)

    def drain_extract(g, slot):
        lm_vec = jax.lax.bitwise_and(idx_m[pl.ds(g * _G, 16)], 127)
        rs_vec = jax.lax.bitwise_and(idx_s[pl.ds(g * _G, 16)], 7)
        for k in range(_G):
            pltpu.make_async_copy(
                wmt_hbm.at[:, pl.ds(0, 128)], blk_m.at[slot, k], sem_m).wait()
        pltpu.make_async_copy(ws_hbm.at[pl.ds(0, _G)], blk_s.at[slot], sem_s).wait()
        for k in range(_G):
            lane = jnp.full((16,), lm_vec[k], jnp.int32)
            rows_v[slot, k, pl.ds(0, 16)] = plsc.load_gather(
                blk_m.at[slot, k], [iota16, lane])
            rows_v[slot, k, pl.ds(16, 16)] = plsc.load_gather(
                blk_m.at[slot, k], [iota16 + 16, lane])
            rows_v[slot, k, pl.ds(_DM, 16)] = blk_s[slot, k, rs_vec[k], pl.ds(0, 16)]
        pltpu.async_copy(rows_v.at[slot], out_hbm.at[pl.ds(base + g * _G, _G)], sem_o)

    fire(0, 0)

    @pl.loop(0, _NG)
    def _body(c):
        slot = jax.lax.bitwise_and(c, 1)
        nslot = jax.lax.bitwise_and(c + 1, 1)

        @pl.when(c + 1 < _NG)
        def _():
            fire(c + 1, nslot)

        @pl.when(c >= 2)
        def _():
            pltpu.make_async_copy(
                rows_v.at[slot], out_hbm.at[pl.ds(base, _G)], sem_o).wait()

        drain_extract(c, slot)

    pltpu.make_async_copy(rows_v.at[0], out_hbm.at[pl.ds(base, _G)], sem_o).wait()
    pltpu.make_async_copy(rows_v.at[1], out_hbm.at[pl.ds(base, _G)], sem_o).wait()


def kernel(material_id, space_group, W_material, W_space):
    wmt = W_material.T
    ws3 = W_space.reshape(12500, 8, _DS)
    return _gather_kernel(material_id.astype(jnp.int32),
                          space_group.astype(jnp.int32), wmt, ws3)
